# Initial kernel scaffold; baseline (speedup 1.0000x reference)
import jax
import jax.numpy as jnp
from jax.experimental import pallas as pl


def kernel(x, gate_w, w1, w2, w3):
    raise NotImplementedError("write your pallas kernel here")


# speedup vs baseline: 1.6823x; 1.6823x over previous
import functools

import jax
import jax.numpy as jnp
from jax.experimental import pallas as pl
from jax.experimental.pallas import tpu as pltpu

_E = 8
_TOPK = 2
_NEG = -1e30


def _dense_body(gw_ref, x_ref, w1_ref, w3_ref, w2_ref, out_ref,
                a0_ref, a1_ref, p0_ref, p1_ref):
    e = pl.program_id(0)
    hh = pl.program_id(1)
    T = x_ref.shape[0]

    @pl.when((e == 0) & (hh == 0))
    def _gate():
        lg = jax.lax.dot_general(
            x_ref[...], gw_ref[...], (((1,), (1,)), ((), ())),
            preferred_element_type=jnp.float32)
        lane = jax.lax.broadcasted_iota(jnp.int32, lg.shape, 1)
        lg = jnp.where(lane < _E, lg, _NEG)
        m0 = jnp.max(lg, axis=1, keepdims=True)
        a0 = jnp.min(jnp.where(lg == m0, lane, 128), axis=1, keepdims=True)
        lg1 = jnp.where(lane == a0, _NEG, lg)
        m1 = jnp.max(lg1, axis=1, keepdims=True)
        a1 = jnp.min(jnp.where(lg1 == m1, lane, 128), axis=1, keepdims=True)
        p0 = 1.0 / (1.0 + jnp.exp(m1 - m0))
        a0_ref[...] = a0
        a1_ref[...] = a1
        p0_ref[...] = p0
        p1_ref[...] = 1.0 - p0
        out_ref[...] = jnp.zeros_like(out_ref)

    we = jnp.where(a0_ref[...] == e, p0_ref[...],
                   jnp.where(a1_ref[...] == e, p1_ref[...], 0.0))

    x = x_ref[...]
    h1 = jax.lax.dot_general(x, w1_ref[0], (((1,), (1,)), ((), ())),
                             preferred_element_type=jnp.float32)
    h3 = jax.lax.dot_general(x, w3_ref[0], (((1,), (1,)), ((), ())),
                             preferred_element_type=jnp.float32)
    h = (h1 / (1.0 + jnp.exp(-h1))) * h3
    yc = jax.lax.dot_general(h, w2_ref[0], (((1,), (1,)), ((), ())),
                             preferred_element_type=jnp.float32)
    out_ref[...] += we * yc


def kernel(x, gate_w, w1, w2, w3):
    Bb, S, D = x.shape
    T = Bb * S
    E, H = w1.shape[0], w1.shape[1]
    HH = 512
    NH = H // HH
    xf = x.reshape(T, D)
    gwp = jnp.zeros((128, D), jnp.float32).at[:E].set(gate_w)

    out = pl.pallas_call(
        _dense_body,
        grid=(E, NH),
        in_specs=[
            pl.BlockSpec((128, D), lambda e, hh: (0, 0)),
            pl.BlockSpec((T, D), lambda e, hh: (0, 0)),
            pl.BlockSpec((1, HH, D), lambda e, hh: (e, hh, 0)),
            pl.BlockSpec((1, HH, D), lambda e, hh: (e, hh, 0)),
            pl.BlockSpec((1, D, HH), lambda e, hh: (e, 0, hh)),
        ],
        out_specs=pl.BlockSpec((T, D), lambda e, hh: (0, 0)),
        out_shape=jax.ShapeDtypeStruct((T, D), jnp.float32),
        scratch_shapes=[
            pltpu.VMEM((T, 1), jnp.int32),
            pltpu.VMEM((T, 1), jnp.int32),
            pltpu.VMEM((T, 1), jnp.float32),
            pltpu.VMEM((T, 1), jnp.float32),
        ],
        compiler_params=pltpu.CompilerParams(
            dimension_semantics=("arbitrary", "arbitrary"),
        ),
    )(gwp, xf, w1, w3, w2)
    return out.reshape(Bb, S, D)
